# initial kernel scaffold (unmeasured)
import jax
import jax.numpy as jnp
from jax import lax
from jax.experimental import pallas as pl
from jax.experimental.pallas import tpu as pltpu


def kernel(
    x,
):
    def body(*refs):
        pass

    out_shape = jax.ShapeDtypeStruct(..., jnp.float32)
    return pl.pallas_call(body, out_shape=out_shape)(...)


# baseline (device time: 13385 ns/iter reference)
import jax
import jax.numpy as jnp
from jax import lax
from jax.experimental import pallas as pl
from jax.experimental.pallas import tpu as pltpu


def kernel(x):
    x2d = x.reshape(x.shape[-2], x.shape[-1])
    m, n = x2d.shape

    def body(x_ref, out_ref, comm_ref, accum_ref, send_sems, recv_sems):
        my_x = lax.axis_index("x")
        my_y = lax.axis_index("y")
        y_nbr = (my_x, 1 - my_y)
        x_nbr = (1 - my_x, my_y)

        barrier_sem = pltpu.get_barrier_semaphore()
        for nbr in (y_nbr, x_nbr):
            pl.semaphore_signal(
                barrier_sem, inc=1,
                device_id=nbr, device_id_type=pl.DeviceIdType.MESH,
            )
        pl.semaphore_wait(barrier_sem, 2)

        rdma_y = pltpu.make_async_remote_copy(
            src_ref=x_ref,
            dst_ref=comm_ref.at[0],
            send_sem=send_sems.at[0],
            recv_sem=recv_sems.at[0],
            device_id=y_nbr,
            device_id_type=pl.DeviceIdType.MESH,
        )
        rdma_y.start()
        rdma_y.wait()
        accum_ref[...] = x_ref[...] + comm_ref[0]

        rdma_x = pltpu.make_async_remote_copy(
            src_ref=accum_ref,
            dst_ref=comm_ref.at[1],
            send_sem=send_sems.at[1],
            recv_sem=recv_sems.at[1],
            device_id=x_nbr,
            device_id_type=pl.DeviceIdType.MESH,
        )
        rdma_x.start()
        rdma_x.wait()
        out_ref[...] = accum_ref[...] + comm_ref[1]

    return pl.pallas_call(
        body,
        out_shape=jax.ShapeDtypeStruct((m, n), jnp.float32),
        in_specs=[pl.BlockSpec(memory_space=pltpu.VMEM)],
        out_specs=pl.BlockSpec(memory_space=pltpu.VMEM),
        scratch_shapes=[
            pltpu.VMEM((2, m, n), jnp.float32),
            pltpu.VMEM((m, n), jnp.float32),
            pltpu.SemaphoreType.DMA((2,)),
            pltpu.SemaphoreType.DMA((2,)),
        ],
        compiler_params=pltpu.CompilerParams(collective_id=0),
    )(x2d)


# device time: 10598 ns/iter; 1.2630x vs baseline; 1.2630x over previous
import jax
import jax.numpy as jnp
from jax import lax
from jax.experimental import pallas as pl
from jax.experimental.pallas import tpu as pltpu


def kernel(x):
    x2d = x.reshape(x.shape[-2], x.shape[-1])
    m, n = x2d.shape
    h = m // 2

    def body(x_ref, out_ref, comm_ref, partial_ref, send_sems, recv_sems):
        my_x = lax.axis_index("x")
        my_y = lax.axis_index("y")
        y_nbr = (my_x, 1 - my_y)
        x_nbr = (1 - my_x, my_y)

        barrier_sem = pltpu.get_barrier_semaphore()
        for nbr in (y_nbr, x_nbr):
            pl.semaphore_signal(
                barrier_sem, inc=1,
                device_id=nbr, device_id_type=pl.DeviceIdType.MESH,
            )
        pl.semaphore_wait(barrier_sem, 2)

        r1a = pltpu.make_async_remote_copy(
            src_ref=x_ref.at[pl.ds(0, h), :],
            dst_ref=comm_ref.at[0],
            send_sem=send_sems.at[0],
            recv_sem=recv_sems.at[0],
            device_id=y_nbr,
            device_id_type=pl.DeviceIdType.MESH,
        )
        r1b = pltpu.make_async_remote_copy(
            src_ref=x_ref.at[pl.ds(h, h), :],
            dst_ref=comm_ref.at[1],
            send_sem=send_sems.at[1],
            recv_sem=recv_sems.at[1],
            device_id=x_nbr,
            device_id_type=pl.DeviceIdType.MESH,
        )
        r1a.start()
        r1b.start()

        r1a.wait_recv()
        partial_ref[0] = x_ref[pl.ds(0, h), :] + comm_ref[0]
        r2a = pltpu.make_async_remote_copy(
            src_ref=partial_ref.at[0],
            dst_ref=comm_ref.at[2],
            send_sem=send_sems.at[2],
            recv_sem=recv_sems.at[2],
            device_id=x_nbr,
            device_id_type=pl.DeviceIdType.MESH,
        )
        r2a.start()

        r1b.wait_recv()
        partial_ref[1] = x_ref[pl.ds(h, h), :] + comm_ref[1]
        r2b = pltpu.make_async_remote_copy(
            src_ref=partial_ref.at[1],
            dst_ref=comm_ref.at[3],
            send_sem=send_sems.at[3],
            recv_sem=recv_sems.at[3],
            device_id=y_nbr,
            device_id_type=pl.DeviceIdType.MESH,
        )
        r2b.start()

        r2a.wait_recv()
        out_ref[pl.ds(0, h), :] = partial_ref[0] + comm_ref[2]
        r2b.wait_recv()
        out_ref[pl.ds(h, h), :] = partial_ref[1] + comm_ref[3]

        r1a.wait_send()
        r1b.wait_send()
        r2a.wait_send()
        r2b.wait_send()

    return pl.pallas_call(
        body,
        out_shape=jax.ShapeDtypeStruct((m, n), jnp.float32),
        in_specs=[pl.BlockSpec(memory_space=pltpu.VMEM)],
        out_specs=pl.BlockSpec(memory_space=pltpu.VMEM),
        scratch_shapes=[
            pltpu.VMEM((4, h, n), jnp.float32),
            pltpu.VMEM((2, h, n), jnp.float32),
            pltpu.SemaphoreType.DMA((4,)),
            pltpu.SemaphoreType.DMA((4,)),
        ],
        compiler_params=pltpu.CompilerParams(collective_id=0),
    )(x2d)
